# baseline (speedup 1.0000x reference)
import functools

import jax
import jax.numpy as jnp
from jax import lax
from jax.experimental import pallas as pl
from jax.experimental.pallas import tpu as pltpu
from jax.experimental.pallas import tpu_sc as plsc

NC, NS, L = 2, 16, 16
CHUNK = 112


def _round_up(x, m):
    return -(-x // m) * m


def _lane_bcast(v, k):
    idx = jnp.full((L, 1), k, jnp.int32)
    dnums = lax.GatherDimensionNumbers(
        offset_dims=(), collapsed_slice_dims=(0,), start_index_map=(0,))
    return lax.gather(v, idx, dnums, slice_sizes=(1,),
                      mode=lax.GatherScatterMode.PROMISE_IN_BOUNDS)


def _make_sc_agg(B, N, D, n_chunks):
    npad = _round_up(N + 1, NS * 8)
    zrows = npad // NS
    mesh = plsc.VectorSubcoreMesh(core_axis_name="c", subcore_axis_name="s")

    @functools.partial(
        pl.kernel,
        out_type=jax.ShapeDtypeStruct((NC, B, npad, D), jnp.float32),
        mesh=mesh,
        compiler_params=pltpu.CompilerParams(needs_layout_passes=False),
        scratch_types=[
            pltpu.VMEM_SHARED((npad, D), jnp.float32),
            pltpu.VMEM((3, 2, CHUNK), jnp.int32),
            pltpu.VMEM((3, CHUNK), jnp.int32),
            pltpu.VMEM((3, CHUNK, D), jnp.float32),
            pltpu.SemaphoreType.DMA((3,)),
            pltpu.SemaphoreType.DMA((3,)),
            pltpu.SemaphoreType.DMA((3,)),
            pltpu.SemaphoreType.DMA((3,)),
        ],
    )
    def sc_agg(nodes_hbm, gw_hbm, sx_hbm, zeros_hbm, out_hbm,
               acc, gbufs, sbufs, rbufs, gsem, ssem, gisem, sisem):
        c = lax.axis_index("c")
        s = lax.axis_index("s")

        for b in range(B):
            pltpu.sync_copy(zeros_hbm, acc.at[pl.ds(s * zrows, zrows)])
            plsc.subcore_barrier()

            def idxg_start(j, i):
                pltpu.async_copy(gw_hbm.at[c, b, s, i], gbufs.at[j],
                                 gisem.at[j])

            def idxg_wait(j):
                pltpu.make_async_copy(gw_hbm.at[c, b, s, 0], gbufs.at[j],
                                      gisem.at[j]).wait()

            def idxs_start(j, i):
                pltpu.async_copy(sx_hbm.at[c, b, s, i], sbufs.at[j],
                                 sisem.at[j])

            def idxs_wait(j):
                pltpu.make_async_copy(sx_hbm.at[c, b, s, 0], sbufs.at[j],
                                      sisem.at[j]).wait()

            def gather_start(j):
                pltpu.async_copy(nodes_hbm.at[gbufs.at[j, 0]],
                                 rbufs.at[j], gsem.at[j])

            def gather_wait(j):
                pltpu.make_async_copy(nodes_hbm.at[gbufs.at[j, 0]],
                                      rbufs.at[j], gsem.at[j]).wait()

            def scat_start(j):
                pltpu.async_copy(rbufs.at[j], acc.at[sbufs.at[j]],
                                 ssem.at[j], add=True)

            def scat_wait(j):
                pltpu.make_async_copy(rbufs.at[j], acc.at[sbufs.at[j]],
                                      ssem.at[j]).wait()

            def scale(j):
                wrow = gbufs.at[j, 1]
                rbuf = rbufs.at[j]

                def blk_body(t, rcarry):
                    r0 = t * L
                    wch = plsc.bitcast(wrow[pl.ds(r0, L)], jnp.float32)
                    for k in range(L):
                        wb = _lane_bcast(wch, k)
                        for g in range(D // L):
                            sl = pl.ds(g * L, L)
                            rbuf[r0 + k, sl] = rbuf[r0 + k, sl] * wb
                    return rcarry

                lax.fori_loop(0, CHUNK // L, blk_body, 0)

            def step(j, jprev, i, prefetch=True):
                if prefetch:
                    idxg_wait(jprev)
                gather_wait(j)
                scale(j)
                idxs_wait(j)
                scat_start(j)
                if prefetch:
                    scat_wait(jprev)
                    idxg_start(j, i + 3)
                    idxs_start(jprev, i + 2)
                    gather_start(jprev)

            idxg_start(0, 0)
            idxg_start(1, 1)
            idxg_start(2, 2)
            idxs_start(0, 0)
            idxs_start(1, 1)
            idxs_start(2, 2)
            idxg_wait(0)
            gather_start(0)
            idxg_wait(1)
            gather_start(1)
            idxg_wait(2)
            gather_wait(0)
            scale(0)
            idxs_wait(0)
            scat_start(0)
            idxg_start(0, 3)
            gather_start(2)
            idxg_wait(0)
            gather_wait(1)
            scale(1)
            idxs_wait(1)
            scat_start(1)
            scat_wait(0)
            idxg_start(1, 4)
            idxs_start(0, 3)
            gather_start(0)

            def body3(g, carry):
                i = 3 * g + 2
                step(2, 1, i)
                step(0, 2, i + 1)
                step(1, 0, i + 2)
                return carry

            n_groups = (n_chunks - 4) // 3
            lax.fori_loop(0, n_groups, body3, 0)

            step(2, 1, 0, prefetch=False)
            scat_wait(1)
            step(0, 2, 0, prefetch=False)
            scat_wait(2)
            scat_wait(0)
            idxg_wait(1)

            plsc.subcore_barrier()
            pltpu.sync_copy(acc.at[pl.ds(s * zrows, zrows)],
                            out_hbm.at[c, b, pl.ds(s * zrows, zrows)])
            plsc.subcore_barrier()

    return sc_agg


def _mlp_body(ai, ao, nd, w1a, w1b, w1c, b1, g1, be1, w2, b2, g2, be2,
              w3, b3, g3, be3, w4, b4, g4, be4, out):
    def ln(x, g, be):
        m = jnp.mean(x, axis=-1, keepdims=True)
        v = jnp.mean((x - m) ** 2, axis=-1, keepdims=True)
        return (x - m) / jnp.sqrt(v + 1e-5) * g + be

    dot = functools.partial(jnp.dot, preferred_element_type=jnp.float32)
    h = (dot(ai[0], w1a[...]) + dot(ao[0], w1b[...]) + dot(nd[0], w1c[...])
         + b1[...])
    h = jnp.tanh(ln(h, g1[...], be1[...]))
    h = jnp.tanh(ln(dot(h, w2[...]) + b2[...], g2[...], be2[...]))
    h = jnp.tanh(ln(dot(h, w3[...]) + b3[...], g3[...], be3[...]))
    h = jnp.tanh(ln(dot(h, w4[...]) + b4[...], g4[...], be4[...]))
    out[0] = h


def _mlp(agg_in, agg_out, nodes, params, row_block):
    B, N, D = nodes.shape
    grid = (B, N // row_block)
    node_spec = pl.BlockSpec((1, row_block, D), lambda b, i: (b, i, 0))
    w_spec = pl.BlockSpec((D, D), lambda b, i: (0, 0))
    v_spec = pl.BlockSpec((1, D), lambda b, i: (0, 0))
    specs = [node_spec] * 3 + [w_spec] * 3 + [v_spec] * 3 + \
        ([w_spec] + [v_spec] * 3) * 3
    return pl.pallas_call(
        _mlp_body,
        grid=grid,
        in_specs=specs,
        out_specs=node_spec,
        out_shape=jax.ShapeDtypeStruct((B, N, D), jnp.float32),
    )(agg_in, agg_out, nodes, *params)


def kernel(nodes, edges, edge_weights, W1, b1, g1, be1, W2, b2, g2, be2,
           W3, b3, g3, be3, W4, b4, g4, be4):
    B, N, D = nodes.shape
    E = edges.shape[1]
    n_chunks = -(-E // (NS * CHUNK))
    while n_chunks % 3 != 1 or n_chunks < 7:
        n_chunks += 1
    e_pad = NS * n_chunks * CHUNK
    pad = e_pad - E

    src = edges[..., 0]
    dst = edges[..., 1]
    offs = (jnp.arange(B, dtype=jnp.int32) * N)[:, None]
    gidx = jnp.stack([src + offs, dst + offs])
    sidx = jnp.stack([dst, src])
    gidx = jnp.pad(gidx, ((0, 0), (0, 0), (0, pad)))
    sidx = jnp.pad(sidx, ((0, 0), (0, 0), (0, pad)), constant_values=N)
    w = jnp.broadcast_to(edge_weights, (NC, B, E))
    w = jnp.pad(w, ((0, 0), (0, 0), (0, pad)))
    wbits = lax.bitcast_convert_type(w, jnp.int32)
    gw = jnp.stack([gidx, wbits], axis=2)
    gw = gw.reshape(NC, B, 2, NS, n_chunks, CHUNK).transpose(0, 1, 3, 4, 2, 5)
    gw = jnp.pad(gw, ((0, 0), (0, 0), (0, 0), (0, 1), (0, 0), (0, 0)))
    sx = sidx.reshape(NC, B, NS, n_chunks, CHUNK)
    nodes_flat = nodes.reshape(B * N, D)
    zeros = jnp.zeros((_round_up(N + 1, NS * 8) // NS, D), jnp.float32)

    agg = _make_sc_agg(B, N, D, n_chunks)(nodes_flat, gw, sx, zeros)

    params = (W1[:D], W1[D:2 * D], W1[2 * D:],
              b1[None], g1[None], be1[None],
              W2, b2[None], g2[None], be2[None],
              W3, b3[None], g3[None], be3[None],
              W4, b4[None], g4[None], be4[None])
    return _mlp(agg[0], agg[1], nodes, params, row_block=400)


# speedup vs baseline: 1.8468x; 1.8468x over previous
import functools

import jax
import jax.numpy as jnp
from jax import lax
from jax.experimental import pallas as pl
from jax.experimental.pallas import tpu as pltpu
from jax.experimental.pallas import tpu_sc as plsc

NC, NS, L = 2, 16, 16
CHUNK = 80
BLK = 12


def _round_up(x, m):
    return -(-x // m) * m


def _lane_bcast(v, k):
    idx = jnp.full((L, 1), k, jnp.int32)
    dnums = lax.GatherDimensionNumbers(
        offset_dims=(), collapsed_slice_dims=(0,), start_index_map=(0,))
    return lax.gather(v, idx, dnums, slice_sizes=(1,),
                      mode=lax.GatherScatterMode.PROMISE_IN_BOUNDS)


def _plan(E):
    n_chunks = -(-E // (NS * CHUNK))
    while n_chunks % 3 != 1 or n_chunks < 13:
        n_chunks += 1
    n_groups = (n_chunks - 4) // 3
    n_cross = len([g for g in range(n_groups) if (3 * g + 6) % BLK == 0])
    last_fetch = n_cross + 1
    n_alloc = (last_fetch + 1) * BLK
    return n_chunks, n_groups, last_fetch, n_alloc


def _make_sc_agg(B, N, D, E):
    n_chunks, n_groups, last_fetch, n_alloc = _plan(E)
    npad = _round_up(N + 1, NS * 8)
    zrows = npad // NS
    mesh = plsc.VectorSubcoreMesh(core_axis_name="c", subcore_axis_name="s")

    @functools.partial(
        pl.kernel,
        out_type=jax.ShapeDtypeStruct((NC, B, npad, D), jnp.float32),
        mesh=mesh,
        compiler_params=pltpu.CompilerParams(needs_layout_passes=False),
        scratch_types=[
            pltpu.VMEM_SHARED((npad, D), jnp.float32),
            pltpu.VMEM((3, BLK, 3, CHUNK), jnp.int32),
            pltpu.VMEM((3, CHUNK, D), jnp.float32),
            pltpu.SemaphoreType.DMA((3,)),
            pltpu.SemaphoreType.DMA((3,)),
            pltpu.SemaphoreType.DMA((3,)),
        ],
    )
    def sc_agg(nodes_hbm, gsw_hbm, zeros_hbm, out_hbm,
               acc, iblk, rbufs, gsem, ssem, bsem):
        c = lax.axis_index("c")
        s = lax.axis_index("s")

        for bt in range(B):
            pltpu.sync_copy(zeros_hbm, acc.at[pl.ds(s * zrows, zrows)])
            plsc.subcore_barrier()

            def idx_ref(i):
                return iblk.at[(i // BLK) % 3, i % BLK]

            def blk_start(m):
                pltpu.async_copy(gsw_hbm.at[c, bt, s, pl.ds(m * BLK, BLK)],
                                 iblk.at[m % 3], bsem.at[m % 3])

            def blk_wait(m):
                pltpu.make_async_copy(
                    gsw_hbm.at[c, bt, s, pl.ds(0, BLK)],
                    iblk.at[m % 3], bsem.at[m % 3]).wait()

            def gather_start(j, i):
                pltpu.async_copy(nodes_hbm.at[idx_ref(i).at[0]],
                                 rbufs.at[j], gsem.at[j])

            def gather_wait(j, i):
                pltpu.make_async_copy(nodes_hbm.at[idx_ref(i).at[0]],
                                      rbufs.at[j], gsem.at[j]).wait()

            def scat_start(j, i):
                pltpu.async_copy(rbufs.at[j], acc.at[idx_ref(i).at[1]],
                                 ssem.at[j], add=True)

            def scat_wait(j, i):
                pltpu.make_async_copy(rbufs.at[j], acc.at[idx_ref(i).at[1]],
                                      ssem.at[j]).wait()

            def scale(j, i):
                wrow = idx_ref(i).at[2]
                rbuf = rbufs.at[j]

                def blk_body(t, rcarry):
                    r0 = t * L
                    wch = plsc.bitcast(wrow[pl.ds(r0, L)], jnp.float32)
                    for k in range(L):
                        wb = _lane_bcast(wch, k)
                        for g in range(D // L):
                            sl = pl.ds(g * L, L)
                            rbuf[r0 + k, sl] = rbuf[r0 + k, sl] * wb
                    return rcarry

                lax.fori_loop(0, CHUNK // L, blk_body, 0)

            def step(j, jprev, i, prefetch=True, cross=False):
                if cross:
                    mnew = (i + 2) // BLK

                    @pl.when((i + 2) % BLK == 0)
                    def _():
                        blk_wait(mnew)
                        blk_start(mnew + 1)

                gather_wait(j, i)
                scale(j, i)
                scat_start(j, i)
                if prefetch:
                    scat_wait(jprev, i - 1)
                    gather_start(jprev, i + 2)

            blk_start(0)
            blk_wait(0)
            blk_start(1)
            gather_start(0, 0)
            gather_start(1, 1)
            step(0, 2, 0, prefetch=False)
            gather_start(2, 2)
            step(1, 0, 1, prefetch=False)
            scat_wait(0, 0)
            gather_start(0, 3)

            def body3(g, carry):
                i = 3 * g + 2
                step(2, 1, i)
                step(0, 2, i + 1)
                step(1, 0, i + 2, cross=True)
                return carry

            lax.fori_loop(0, n_groups, body3, 0)

            nl = n_chunks - 2
            step(2, 1, nl, prefetch=False)
            scat_wait(1, nl - 1)
            step(0, 2, nl + 1, prefetch=False)
            scat_wait(2, nl)
            scat_wait(0, nl + 1)
            blk_wait(last_fetch)

            plsc.subcore_barrier()
            pltpu.sync_copy(acc.at[pl.ds(s * zrows, zrows)],
                            out_hbm.at[c, bt, pl.ds(s * zrows, zrows)])
            plsc.subcore_barrier()

    return sc_agg


def _mlp_body(ai, ao, nd, w1a, w1b, w1c, b1, g1, be1, w2, b2, g2, be2,
              w3, b3, g3, be3, w4, b4, g4, be4, out):
    def ln(x, g, be):
        m = jnp.mean(x, axis=-1, keepdims=True)
        v = jnp.mean((x - m) ** 2, axis=-1, keepdims=True)
        return (x - m) / jnp.sqrt(v + 1e-5) * g + be

    dot = functools.partial(jnp.dot, preferred_element_type=jnp.float32)
    h = (dot(ai[0], w1a[...]) + dot(ao[0], w1b[...]) + dot(nd[0], w1c[...])
         + b1[...])
    h = jnp.tanh(ln(h, g1[...], be1[...]))
    h = jnp.tanh(ln(dot(h, w2[...]) + b2[...], g2[...], be2[...]))
    h = jnp.tanh(ln(dot(h, w3[...]) + b3[...], g3[...], be3[...]))
    h = jnp.tanh(ln(dot(h, w4[...]) + b4[...], g4[...], be4[...]))
    out[0] = h


def _mlp(agg_in, agg_out, nodes, params, row_block):
    B, N, D = nodes.shape
    grid = (B, N // row_block)
    node_spec = pl.BlockSpec((1, row_block, D), lambda b, i: (b, i, 0))
    w_spec = pl.BlockSpec((D, D), lambda b, i: (0, 0))
    v_spec = pl.BlockSpec((1, D), lambda b, i: (0, 0))
    specs = [node_spec] * 3 + [w_spec] * 3 + [v_spec] * 3 + \
        ([w_spec] + [v_spec] * 3) * 3
    return pl.pallas_call(
        _mlp_body,
        grid=grid,
        in_specs=specs,
        out_specs=node_spec,
        out_shape=jax.ShapeDtypeStruct((B, N, D), jnp.float32),
    )(agg_in, agg_out, nodes, *params)


def kernel(nodes, edges, edge_weights, W1, b1, g1, be1, W2, b2, g2, be2,
           W3, b3, g3, be3, W4, b4, g4, be4):
    B, N, D = nodes.shape
    E = edges.shape[1]
    n_chunks, _, _, n_alloc = _plan(E)
    e_pad = NS * n_chunks * CHUNK
    pad = e_pad - E

    src = edges[..., 0]
    dst = edges[..., 1]
    offs = (jnp.arange(B, dtype=jnp.int32) * N)[:, None]
    gidx = jnp.stack([src + offs, dst + offs])
    sidx = jnp.stack([dst, src])
    gidx = jnp.pad(gidx, ((0, 0), (0, 0), (0, pad)))
    sidx = jnp.pad(sidx, ((0, 0), (0, 0), (0, pad)), constant_values=N)
    w = jnp.broadcast_to(edge_weights, (NC, B, E))
    w = jnp.pad(w, ((0, 0), (0, 0), (0, pad)))
    wbits = lax.bitcast_convert_type(w, jnp.int32)
    gsw = jnp.stack([gidx, sidx, wbits], axis=2)
    gsw = gsw.reshape(NC, B, 3, NS, n_chunks, CHUNK).transpose(0, 1, 3, 4, 2, 5)
    gsw = jnp.pad(
        gsw, ((0, 0), (0, 0), (0, 0), (0, n_alloc - n_chunks), (0, 0), (0, 0)))
    nodes_flat = nodes.reshape(B * N, D)
    zeros = jnp.zeros((_round_up(N + 1, NS * 8) // NS, D), jnp.float32)

    agg = _make_sc_agg(B, N, D, E)(nodes_flat, gsw, zeros)

    params = (W1[:D], W1[D:2 * D], W1[2 * D:],
              b1[None], g1[None], be1[None],
              W2, b2[None], g2[None], be2[None],
              W3, b3[None], g3[None], be3[None],
              W4, b4[None], g4[None], be4[None])
    return _mlp(agg[0], agg[1], nodes, params, row_block=400)


# speedup vs baseline: 1.9978x; 1.0817x over previous
import functools

import jax
import jax.numpy as jnp
from jax import lax
from jax.experimental import pallas as pl
from jax.experimental.pallas import tpu as pltpu
from jax.experimental.pallas import tpu_sc as plsc

NC, NS, L = 2, 16, 16
CHUNK = 80
BLK = 12


def _round_up(x, m):
    return -(-x // m) * m


def _lane_bcast(v, k):
    idx = jnp.full((L, 1), k, jnp.int32)
    dnums = lax.GatherDimensionNumbers(
        offset_dims=(), collapsed_slice_dims=(0,), start_index_map=(0,))
    return lax.gather(v, idx, dnums, slice_sizes=(1,),
                      mode=lax.GatherScatterMode.PROMISE_IN_BOUNDS)


def _plan(E):
    n_chunks = -(-E // (NS * CHUNK))
    while n_chunks % 3 != 1 or n_chunks < 13:
        n_chunks += 1
    n_groups = (n_chunks - 4) // 3
    n_cross = len([g for g in range(n_groups) if (3 * g + 6) % BLK == 0])
    last_fetch = n_cross + 1
    n_alloc = (last_fetch + 1) * BLK
    return n_chunks, n_groups, last_fetch, n_alloc


def _make_sc_agg(B, N, D, E):
    n_chunks, n_groups, last_fetch, n_alloc = _plan(E)
    npad = _round_up(N + 1, NS * 8)
    zrows = npad // NS
    mesh = plsc.VectorSubcoreMesh(core_axis_name="c", subcore_axis_name="s")

    @functools.partial(
        pl.kernel,
        out_type=jax.ShapeDtypeStruct((NC, B, npad, D), jnp.float32),
        mesh=mesh,
        compiler_params=pltpu.CompilerParams(needs_layout_passes=False),
        scratch_types=[
            pltpu.VMEM_SHARED((npad, D), jnp.float32),
            pltpu.VMEM((3, BLK, 3, CHUNK), jnp.int32),
            pltpu.VMEM((3, CHUNK, D), jnp.float32),
            pltpu.SemaphoreType.DMA((3,)),
            pltpu.SemaphoreType.DMA((3,)),
            pltpu.SemaphoreType.DMA((3,)),
        ],
    )
    def sc_agg(nodes_hbm, gsw_hbm, zeros_hbm, out_hbm,
               acc, iblk, rbufs, gsem, ssem, bsem):
        c = lax.axis_index("c")
        s = lax.axis_index("s")

        for bt in range(B):
            pltpu.sync_copy(zeros_hbm, acc.at[pl.ds(s * zrows, zrows)])
            plsc.subcore_barrier()

            def idx_ref(i):
                return iblk.at[(i // BLK) % 3, i % BLK]

            def blk_start(m):
                pltpu.async_copy(gsw_hbm.at[c, bt, s, pl.ds(m * BLK, BLK)],
                                 iblk.at[m % 3], bsem.at[m % 3])

            def blk_wait(m):
                pltpu.make_async_copy(
                    gsw_hbm.at[c, bt, s, pl.ds(0, BLK)],
                    iblk.at[m % 3], bsem.at[m % 3]).wait()

            def gather_start(j, i):
                pltpu.async_copy(nodes_hbm.at[idx_ref(i).at[0]],
                                 rbufs.at[j], gsem.at[j])

            def gather_wait(j, i):
                pltpu.make_async_copy(nodes_hbm.at[idx_ref(i).at[0]],
                                      rbufs.at[j], gsem.at[j]).wait()

            def scat_start(j, i):
                pltpu.async_copy(rbufs.at[j], acc.at[idx_ref(i).at[1]],
                                 ssem.at[j], add=True)

            def scat_wait(j, i):
                pltpu.make_async_copy(rbufs.at[j], acc.at[idx_ref(i).at[1]],
                                      ssem.at[j]).wait()

            def scale(j, i):
                wrow = idx_ref(i).at[2]
                rbuf = rbufs.at[j]

                def blk_body(t, rcarry):
                    r0 = t * L
                    wch = plsc.bitcast(wrow[pl.ds(r0, L)], jnp.float32)
                    for k in range(L):
                        wb = _lane_bcast(wch, k)
                        for g in range(D // L):
                            sl = pl.ds(g * L, L)
                            rbuf[r0 + k, sl] = rbuf[r0 + k, sl] * wb
                    return rcarry

                lax.fori_loop(0, CHUNK // L, blk_body, 0)

            def step(j, jprev, i, prefetch=True, cross=False):
                if cross:
                    mnew = (i + 2) // BLK

                    @pl.when((i + 2) % BLK == 0)
                    def _():
                        blk_wait(mnew)
                        blk_start(mnew + 1)

                gather_wait(j, i)
                scale(j, i)
                scat_start(j, i)
                if prefetch:
                    scat_wait(jprev, i - 1)
                    gather_start(jprev, i + 2)

            blk_start(0)
            blk_wait(0)
            blk_start(1)
            gather_start(0, 0)
            gather_start(1, 1)
            step(0, 2, 0, prefetch=False)
            gather_start(2, 2)
            step(1, 0, 1, prefetch=False)
            scat_wait(0, 0)
            gather_start(0, 3)

            def body3(g, carry):
                i = 3 * g + 2
                step(2, 1, i)
                step(0, 2, i + 1)
                step(1, 0, i + 2, cross=True)
                return carry

            lax.fori_loop(0, n_groups, body3, 0)

            nl = n_chunks - 2
            step(2, 1, nl, prefetch=False)
            scat_wait(1, nl - 1)
            step(0, 2, nl + 1, prefetch=False)
            scat_wait(2, nl)
            scat_wait(0, nl + 1)
            blk_wait(last_fetch)

            plsc.subcore_barrier()
            pltpu.sync_copy(acc.at[pl.ds(s * zrows, zrows)],
                            out_hbm.at[c, bt, pl.ds(s * zrows, zrows)])
            plsc.subcore_barrier()

    return sc_agg


def _mlp_body(ai, ao, nd, w1a, w1b, w1c, b1, g1, be1, w2, b2, g2, be2,
              w3, b3, g3, be3, w4, b4, g4, be4, out):
    def ln(x, g, be):
        m = jnp.mean(x, axis=-1, keepdims=True)
        v = jnp.mean((x - m) ** 2, axis=-1, keepdims=True)
        return (x - m) / jnp.sqrt(v + 1e-5) * g + be

    dot = functools.partial(jnp.dot, preferred_element_type=jnp.float32)
    h = (dot(ai[0, 0], w1a[...]) + dot(ao[0, 0], w1b[...])
         + dot(nd[0], w1c[...]) + b1[...])
    h = jnp.tanh(ln(h, g1[...], be1[...]))
    h = jnp.tanh(ln(dot(h, w2[...]) + b2[...], g2[...], be2[...]))
    h = jnp.tanh(ln(dot(h, w3[...]) + b3[...], g3[...], be3[...]))
    h = jnp.tanh(ln(dot(h, w4[...]) + b4[...], g4[...], be4[...]))
    out[0] = h


def _mlp(agg, nodes, params, row_block):
    B, N, D = nodes.shape
    grid = (B, N // row_block)
    agg_in_spec = pl.BlockSpec((1, 1, row_block, D), lambda b, i: (0, b, i, 0))
    agg_out_spec = pl.BlockSpec((1, 1, row_block, D), lambda b, i: (1, b, i, 0))
    node_spec = pl.BlockSpec((1, row_block, D), lambda b, i: (b, i, 0))
    w_spec = pl.BlockSpec((D, D), lambda b, i: (0, 0))
    v_spec = pl.BlockSpec((1, D), lambda b, i: (0, 0))
    specs = [agg_in_spec, agg_out_spec, node_spec] + [w_spec] * 3 + \
        [v_spec] * 3 + ([w_spec] + [v_spec] * 3) * 3
    return pl.pallas_call(
        _mlp_body,
        grid=grid,
        in_specs=specs,
        out_specs=node_spec,
        out_shape=jax.ShapeDtypeStruct((B, N, D), jnp.float32),
    )(agg, agg, nodes, *params)


def kernel(nodes, edges, edge_weights, W1, b1, g1, be1, W2, b2, g2, be2,
           W3, b3, g3, be3, W4, b4, g4, be4):
    B, N, D = nodes.shape
    E = edges.shape[1]
    n_chunks, _, _, n_alloc = _plan(E)
    e_pad = NS * n_chunks * CHUNK
    pad = e_pad - E

    src = edges[..., 0]
    dst = edges[..., 1]
    offs = (jnp.arange(B, dtype=jnp.int32) * N)[:, None]
    gidx = jnp.stack([src + offs, dst + offs])
    sidx = jnp.stack([dst, src])
    gidx = jnp.pad(gidx, ((0, 0), (0, 0), (0, pad)))
    sidx = jnp.pad(sidx, ((0, 0), (0, 0), (0, pad)), constant_values=N)
    w = jnp.broadcast_to(edge_weights, (NC, B, E))
    w = jnp.pad(w, ((0, 0), (0, 0), (0, pad)))
    wbits = lax.bitcast_convert_type(w, jnp.int32)
    shp = (NC, B, NS, n_chunks, CHUNK)
    gsw = jnp.stack([gidx.reshape(shp), sidx.reshape(shp),
                     wbits.reshape(shp)], axis=4)
    gsw = jnp.pad(
        gsw, ((0, 0), (0, 0), (0, 0), (0, n_alloc - n_chunks), (0, 0), (0, 0)))
    nodes_flat = nodes.reshape(B * N, D)
    zeros = jnp.zeros((_round_up(N + 1, NS * 8) // NS, D), jnp.float32)

    agg = _make_sc_agg(B, N, D, E)(nodes_flat, gsw, zeros)

    params = (W1[:D], W1[D:2 * D], W1[2 * D:],
              b1[None], g1[None], be1[None],
              W2, b2[None], g2[None], be2[None],
              W3, b3[None], g3[None], be3[None],
              W4, b4[None], g4[None], be4[None])
    return _mlp(agg, nodes, params, row_block=1000)
